# baseline (device time: 18957 ns/iter reference)
import jax
import jax.numpy as jnp
from jax import lax
from jax.experimental import pallas as pl
from jax.experimental.pallas import tpu as pltpu

N_DEV = 8
B = 2
SQ = 256
HALO = 128
HQ = 4
DH = 64
DM = 512


def kernel(x, Wq, K_ext, V_ext, Wo):
    Kt = jnp.transpose(K_ext, (0, 2, 1, 3)).astype(jnp.bfloat16)
    Vt = jnp.transpose(V_ext, (0, 2, 1, 3)).astype(jnp.bfloat16)
    Wqt = (jnp.transpose(Wq.reshape(DM, HQ, DH), (1, 0, 2)) * 0.125
           ).astype(jnp.bfloat16)
    Wo16 = Wo.astype(jnp.bfloat16)

    def body(x_hbm, wq_hbm, k_hbm, v_hbm, wo_hbm, out_ref,
             x_v, wq_v, k_v, v_v, wo_v, khalo, vhalo,
             load_sems, send_sems, recv_sems):
        my = lax.axis_index("i")
        left = jnp.maximum(my - 1, 0)
        right = jnp.minimum(my + 1, N_DEV - 1)

        cp_k = pltpu.make_async_copy(k_hbm, k_v, load_sems.at[0])
        cp_v = pltpu.make_async_copy(v_hbm, v_v, load_sems.at[1])
        cp_x = pltpu.make_async_copy(x_hbm, x_v, load_sems.at[2])
        cp_wq = pltpu.make_async_copy(wq_hbm, wq_v, load_sems.at[3])
        cp_wo = pltpu.make_async_copy(wo_hbm, wo_v, load_sems.at[4])
        cp_k.start()
        cp_v.start()
        cp_x.start()
        cp_wq.start()
        cp_wo.start()

        barrier_sem = pltpu.get_barrier_semaphore()

        @pl.when(my > 0)
        def _():
            pl.semaphore_signal(barrier_sem, inc=1, device_id=(left,),
                                device_id_type=pltpu.DeviceIdType.MESH)

        @pl.when(my < N_DEV - 1)
        def _():
            pl.semaphore_signal(barrier_sem, inc=1, device_id=(right,),
                                device_id_type=pltpu.DeviceIdType.MESH)

        @pl.when(my == 0)
        def _():
            vhalo[:, :, 0:HALO] = jnp.zeros((B, HQ, HALO, DH), jnp.bfloat16)

        @pl.when(my == N_DEV - 1)
        def _():
            vhalo[:, :, HALO:2 * HALO] = jnp.zeros((B, HQ, HALO, DH),
                                                   jnp.bfloat16)

        n_nbrs = (my > 0).astype(jnp.int32) + (my < N_DEV - 1).astype(jnp.int32)
        pl.semaphore_wait(barrier_sem, n_nbrs)
        cp_k.wait()
        cp_v.wait()

        rdma_r_k = pltpu.make_async_remote_copy(
            src_ref=k_v.at[:, :, pl.ds(SQ - HALO, HALO)],
            dst_ref=khalo.at[:, :, pl.ds(0, HALO)],
            send_sem=send_sems.at[0], recv_sem=recv_sems.at[0],
            device_id=(right,), device_id_type=pltpu.DeviceIdType.MESH,
        )
        rdma_l_k = pltpu.make_async_remote_copy(
            src_ref=k_v.at[:, :, pl.ds(0, HALO)],
            dst_ref=khalo.at[:, :, pl.ds(HALO, HALO)],
            send_sem=send_sems.at[1], recv_sem=recv_sems.at[1],
            device_id=(left,), device_id_type=pltpu.DeviceIdType.MESH,
        )
        rdma_r_v = pltpu.make_async_remote_copy(
            src_ref=v_v.at[:, :, pl.ds(SQ - HALO, HALO)],
            dst_ref=vhalo.at[:, :, pl.ds(0, HALO)],
            send_sem=send_sems.at[2], recv_sem=recv_sems.at[2],
            device_id=(right,), device_id_type=pltpu.DeviceIdType.MESH,
        )
        rdma_l_v = pltpu.make_async_remote_copy(
            src_ref=v_v.at[:, :, pl.ds(0, HALO)],
            dst_ref=vhalo.at[:, :, pl.ds(HALO, HALO)],
            send_sem=send_sems.at[3], recv_sem=recv_sems.at[3],
            device_id=(left,), device_id_type=pltpu.DeviceIdType.MESH,
        )

        @pl.when(my < N_DEV - 1)
        def _():
            rdma_r_k.start()

        @pl.when(my > 0)
        def _():
            rdma_l_k.start()

        @pl.when(my < N_DEV - 1)
        def _():
            rdma_r_v.start()

        @pl.when(my > 0)
        def _():
            rdma_l_v.start()

        cp_x.wait()
        cp_wq.wait()

        r_a = lax.broadcasted_iota(jnp.int32, (SQ, SQ), 0)
        j_a = lax.broadcasted_iota(jnp.int32, (SQ, SQ), 1)
        mask_a = jnp.abs(j_a - r_a) <= HALO

        q = []
        ctx = []
        lsum = []
        for b in range(B):
            xb16 = x_v[b].astype(jnp.bfloat16)
            q.append([])
            ctx.append([])
            lsum.append([])
            for h in range(HQ):
                qh = jnp.dot(xb16, wq_v[h],
                             preferred_element_type=jnp.float32)
                qh16 = qh.astype(jnp.bfloat16)
                s = lax.dot_general(
                    qh16, k_v[b, h], (((1,), (1,)), ((), ())),
                    preferred_element_type=jnp.float32,
                )
                w = jnp.where(mask_a, jnp.exp(s), 0.0)
                ctx_h = jnp.dot(w.astype(jnp.bfloat16), v_v[b, h],
                                preferred_element_type=jnp.float32)
                q[b].append(qh16)
                ctx[b].append(ctx_h)
                lsum[b].append(jnp.sum(w, axis=1, keepdims=True))

        @pl.when(my > 0)
        def _():
            rdma_r_k.wait_recv()

        @pl.when(my < N_DEV - 1)
        def _():
            rdma_l_k.wait_recv()

        r_h = lax.broadcasted_iota(jnp.int32, (SQ, 2 * HALO), 0)
        j_h = lax.broadcasted_iota(jnp.int32, (SQ, 2 * HALO), 1)
        mask_h = ((j_h < HALO) & (j_h >= r_h) & (my > 0)) | (
            (j_h >= HALO) & (j_h <= r_h) & (my < N_DEV - 1))

        wh = []
        for b in range(B):
            wh.append([])
            for h in range(HQ):
                s_h = lax.dot_general(
                    q[b][h], khalo[b, h], (((1,), (1,)), ((), ())),
                    preferred_element_type=jnp.float32,
                )
                wh[b].append(jnp.where(mask_h, jnp.exp(s_h), 0.0))

        @pl.when(my > 0)
        def _():
            rdma_r_v.wait_recv()

        @pl.when(my < N_DEV - 1)
        def _():
            rdma_l_v.wait_recv()

        @pl.when(my < N_DEV - 1)
        def _():
            rdma_r_k.wait_send()
            rdma_r_v.wait_send()

        @pl.when(my > 0)
        def _():
            rdma_l_k.wait_send()
            rdma_l_v.wait_send()

        cp_wo.wait()
        for b in range(B):
            acc = jnp.zeros((SQ, DM), jnp.float32)
            for h in range(HQ):
                w_h = wh[b][h]
                ctx_h = ctx[b][h] + jnp.dot(
                    w_h.astype(jnp.bfloat16), vhalo[b, h],
                    preferred_element_type=jnp.float32,
                )
                l_h = lsum[b][h] + jnp.sum(w_h, axis=1, keepdims=True)
                ctx_h = ctx_h / l_h
                acc += jnp.dot(ctx_h.astype(jnp.bfloat16),
                               wo_v[h * DH:(h + 1) * DH, :],
                               preferred_element_type=jnp.float32)
            out_ref[b] = acc

    return pl.pallas_call(
        body,
        out_shape=jax.ShapeDtypeStruct(x.shape, jnp.float32),
        in_specs=[pl.BlockSpec(memory_space=pl.ANY)] * 5,
        out_specs=pl.BlockSpec(memory_space=pltpu.VMEM),
        compiler_params=pltpu.CompilerParams(collective_id=0),
        scratch_shapes=[
            pltpu.VMEM((B, SQ, DM), jnp.float32),
            pltpu.VMEM((HQ, DM, DH), jnp.bfloat16),
            pltpu.VMEM((B, HQ, SQ, DH), jnp.bfloat16),
            pltpu.VMEM((B, HQ, SQ, DH), jnp.bfloat16),
            pltpu.VMEM((HQ * DH, DM), jnp.bfloat16),
            pltpu.VMEM((B, HQ, 2 * HALO, DH), jnp.bfloat16),
            pltpu.VMEM((B, HQ, 2 * HALO, DH), jnp.bfloat16),
            pltpu.SemaphoreType.DMA((5,)),
            pltpu.SemaphoreType.DMA((4,)),
            pltpu.SemaphoreType.DMA((4,)),
        ],
    )(x, Wqt, Kt, Vt, Wo16)
